# baseline (speedup 1.0000x reference)
import functools

import jax
import jax.numpy as jnp
from jax import lax
from jax.experimental import pallas as pl
from jax.experimental.pallas import tpu as pltpu
from jax.experimental.pallas import tpu_sc as plsc

NUM_EXPERTS = 8
TOP_K = 2
HIDDEN = 1024
INTER = 1024
TOKENS = 2048
TM = 256
NASSIGN = TOKENS * TOP_K
NBUF = NASSIGN + NUM_EXPERTS * TM
NTILES = NBUF // TM


def _routing_body(logits_ref, idx_ref, w_ref):
    logits = logits_ref[...]
    m = jnp.max(logits, axis=-1, keepdims=True)
    ex = jnp.exp(logits - m)
    probs = ex / jnp.sum(ex, axis=-1, keepdims=True)
    lanes = lax.broadcasted_iota(jnp.int32, probs.shape, 1)
    m1 = jnp.max(probs, axis=-1, keepdims=True)
    i1 = jnp.min(jnp.where(probs == m1, lanes, NUM_EXPERTS), axis=-1, keepdims=True)
    masked = jnp.where(lanes == i1, -jnp.inf, probs)
    m2 = jnp.max(masked, axis=-1, keepdims=True)
    i2 = jnp.min(jnp.where(masked == m2, lanes, NUM_EXPERTS), axis=-1, keepdims=True)
    denom = m1 + m2
    idx_ref[...] = jnp.where(lanes == 0, i1, jnp.where(lanes == 1, i2, 0))
    w_ref[...] = jnp.where(lanes == 0, m1 / denom, jnp.where(lanes == 1, m2 / denom, 0.0))


def _routing(router_logits):
    return pl.pallas_call(
        _routing_body,
        out_shape=(
            jax.ShapeDtypeStruct((TOKENS, NUM_EXPERTS), jnp.int32),
            jax.ShapeDtypeStruct((TOKENS, NUM_EXPERTS), jnp.float32),
        ),
    )(router_logits)


def _ffn_body(te_ref, xs_ref, w31_ref, w2_ref, wb_ref, out_ref):
    xs = xs_ref[...]
    proj = lax.dot_general(
        xs, w31_ref[0], (((1,), (1,)), ((), ())), preferred_element_type=jnp.float32
    )
    up = proj[:, :INTER]
    gate = proj[:, INTER:]
    h = gate * jax.nn.sigmoid(gate) * up
    y = lax.dot_general(
        h, w2_ref[0], (((1,), (1,)), ((), ())), preferred_element_type=jnp.float32
    )
    out_ref[...] = y * wb_ref[...]


def _ffn(xs, w3_w1_weight, w2_weight, wbuf, tile_eid):
    grid_spec = pltpu.PrefetchScalarGridSpec(
        num_scalar_prefetch=1,
        grid=(NTILES,),
        in_specs=[
            pl.BlockSpec((TM, HIDDEN), lambda i, te: (i, 0)),
            pl.BlockSpec((1, 2 * INTER, HIDDEN), lambda i, te: (te[i], 0, 0)),
            pl.BlockSpec((1, HIDDEN, INTER), lambda i, te: (te[i], 0, 0)),
            pl.BlockSpec((TM, 1), lambda i, te: (i, 0)),
        ],
        out_specs=pl.BlockSpec((TM, HIDDEN), lambda i, te: (i, 0)),
    )
    return pl.pallas_call(
        _ffn_body,
        grid_spec=grid_spec,
        out_shape=jax.ShapeDtypeStruct((NBUF, HIDDEN), jnp.float32),
        compiler_params=pltpu.CompilerParams(dimension_semantics=("arbitrary",)),
    )(tile_eid, xs, w3_w1_weight, w2_weight, wbuf.reshape(NBUF, 1))


def _dispatch(topk_idx, topk_w):
    e_flat = topk_idx[:, :TOP_K].reshape(-1)
    w_flat = topk_w[:, :TOP_K].reshape(-1)
    onehot = (e_flat[:, None] == jnp.arange(NUM_EXPERTS)[None, :]).astype(jnp.int32)
    csum = jnp.cumsum(onehot, axis=0)
    counts = csum[-1]
    rank = jnp.take_along_axis(csum, e_flat[:, None], axis=1)[:, 0] - 1
    padded = ((counts + TM - 1) // TM) * TM
    seg_end = jnp.cumsum(padded)
    offs = seg_end - padded
    slot = offs[e_flat] + rank
    src_tok = jnp.zeros((NBUF,), jnp.int32).at[slot].set(
        jnp.arange(NASSIGN, dtype=jnp.int32) // TOP_K)
    wbuf = jnp.zeros((NBUF,), jnp.float32).at[slot].set(w_flat)
    pos = slot.reshape(TOKENS, TOP_K)
    tile_starts = jnp.arange(NTILES, dtype=jnp.int32) * TM
    tile_eid = jnp.minimum(
        jnp.searchsorted(seg_end, tile_starts, side="right").astype(jnp.int32),
        NUM_EXPERTS - 1)
    return src_tok, wbuf, pos, tile_eid


def _sc_gather(x, src_tok):
    info = plsc.get_sparse_core_info()
    nw = info.num_cores * info.num_subcores
    b_per_w = NBUF // nw
    B = 16
    S = 6
    NB = b_per_w // B
    mesh = plsc.VectorSubcoreMesh(core_axis_name="c", subcore_axis_name="s")

    @functools.partial(
        pl.kernel,
        mesh=mesh,
        out_type=jax.ShapeDtypeStruct((NBUF, HIDDEN), jnp.float32),
        scratch_types=(
            [pltpu.VMEM((b_per_w,), jnp.int32)]
            + [pltpu.VMEM((B, HIDDEN), jnp.float32)] * S
            + [pltpu.SemaphoreType.DMA] * (2 * S)
        ),
    )
    def k(x_hbm, idx_hbm, out_hbm, idx_v, *rest):
        bufs = rest[:S]
        gsem = rest[S:2 * S]
        wsem = rest[2 * S:]
        wid = lax.axis_index("s") * info.num_cores + lax.axis_index("c")
        base = wid * b_per_w
        pltpu.sync_copy(idx_hbm.at[pl.ds(base, b_per_w)], idx_v)
        g_h = [None] * NB
        w_h = [None] * NB
        for s in range(S):
            g_h[s] = pltpu.async_copy(
                x_hbm.at[idx_v[pl.ds(s * B, B)]], bufs[s], gsem[s])
        for b in range(NB):
            s = b % S
            g_h[b].wait()
            w_h[b] = pltpu.async_copy(
                bufs[s], out_hbm.at[pl.ds(base + b * B, B)], wsem[s])
            nb = b + S
            if nb < NB:
                w_h[b].wait()
                g_h[nb] = pltpu.async_copy(
                    x_hbm.at[idx_v[pl.ds(nb * B, B)]], bufs[s], gsem[s])
        for b in range(NB - S, NB):
            w_h[b].wait()

    return k(x, src_tok)


def _sc_combine(ysw, pos0, pos1):
    info = plsc.get_sparse_core_info()
    nw = info.num_cores * info.num_subcores
    t_per_w = TOKENS // nw
    chunk = 32
    n_chunks = t_per_w // chunk
    mesh = plsc.VectorSubcoreMesh(core_axis_name="c", subcore_axis_name="s")

    @functools.partial(
        pl.kernel,
        mesh=mesh,
        out_type=jax.ShapeDtypeStruct((TOKENS, HIDDEN), jnp.float32),
        scratch_types=[
            pltpu.VMEM((chunk,), jnp.int32),
            pltpu.VMEM((chunk,), jnp.int32),
            pltpu.VMEM((chunk, HIDDEN), jnp.float32),
            pltpu.VMEM((chunk, HIDDEN), jnp.float32),
            pltpu.SemaphoreType.DMA,
            pltpu.SemaphoreType.DMA,
        ],
    )
    def k(ysw_hbm, p0_hbm, p1_hbm, out_hbm, i0_v, i1_v, r0_v, r1_v, s0, s1):
        wid = lax.axis_index("s") * info.num_cores + lax.axis_index("c")
        base = wid * t_per_w

        def body(c, _):
            off = base + c * chunk
            pltpu.sync_copy(p0_hbm.at[pl.ds(off, chunk)], i0_v)
            pltpu.sync_copy(p1_hbm.at[pl.ds(off, chunk)], i1_v)
            cp0 = pltpu.async_copy(ysw_hbm.at[i0_v], r0_v, s0)
            cp1 = pltpu.async_copy(ysw_hbm.at[i1_v], r1_v, s1)
            cp0.wait()
            cp1.wait()

            def row(i, _):
                for j in range(HIDDEN // 16):
                    sl = pl.ds(j * 16, 16)
                    r0_v[i, sl] = r0_v[i, sl] + r1_v[i, sl]
                return 0

            lax.fori_loop(0, chunk, row, 0)
            pltpu.sync_copy(r0_v, out_hbm.at[pl.ds(off, chunk)])
            return 0

        lax.fori_loop(0, n_chunks, body, 0)

    return k(ysw, pos0, pos1)


def kernel(x, router_logits, w3_w1_weight, w2_weight):
    topk_idx, topk_w = _routing(router_logits)
    src_tok, wbuf, pos, tile_eid = _dispatch(topk_idx, topk_w)
    xs = _sc_gather(x, src_tok)
    ysw = _ffn(xs, w3_w1_weight, w2_weight, wbuf, tile_eid)
    out = _sc_combine(ysw, pos[:, 0].copy(), pos[:, 1].copy())
    return out.astype(x.dtype)


# speedup vs baseline: 2.1882x; 2.1882x over previous
import functools

import jax
import jax.numpy as jnp
from jax import lax
from jax.experimental import pallas as pl
from jax.experimental.pallas import tpu as pltpu
from jax.experimental.pallas import tpu_sc as plsc

NUM_EXPERTS = 8
TOP_K = 2
HIDDEN = 1024
INTER = 1024
TOKENS = 2048
TM = 256
NASSIGN = TOKENS * TOP_K
NBUF = NASSIGN + NUM_EXPERTS * TM
NTILES = NBUF // TM
CHUNK = 128


def _routing_body(logits_ref, pos_ref, w_ref, teid_ref):
    logits = logits_ref[...]
    m = jnp.max(logits, axis=-1, keepdims=True)
    ex = jnp.exp(logits - m)
    probs = ex / jnp.sum(ex, axis=-1, keepdims=True)
    lanes = lax.broadcasted_iota(jnp.int32, probs.shape, 1)
    m1 = jnp.max(probs, axis=-1, keepdims=True)
    i1 = jnp.min(jnp.where(probs == m1, lanes, NUM_EXPERTS), axis=-1, keepdims=True)
    oh1 = (lanes == i1).astype(jnp.float32)
    masked = jnp.where(lanes == i1, -jnp.inf, probs)
    m2 = jnp.max(masked, axis=-1, keepdims=True)
    i2 = jnp.min(jnp.where(masked == m2, lanes, NUM_EXPERTS), axis=-1, keepdims=True)
    oh2 = (lanes == i2).astype(jnp.float32)
    denom = m1 + m2
    w_ref[...] = jnp.where(lanes == 0, m1 / denom, jnp.where(lanes == 1, m2 / denom, 0.0))

    a = oh1 + oh2
    rr = lax.broadcasted_iota(jnp.int32, (CHUNK, CHUNK), 0)
    cc = lax.broadcasted_iota(jnp.int32, (CHUNK, CHUNK), 1)
    ltri = (cc < rr).astype(jnp.float32)
    ones_row = jnp.ones((1, CHUNK), jnp.float32)
    n_chunks = TOKENS // CHUNK
    pieces = []
    off = jnp.zeros((1, NUM_EXPERTS), jnp.float32)
    for g in range(n_chunks):
        a_g = a[g * CHUNK:(g + 1) * CHUNK, :]
        c_g = lax.dot_general(ltri, a_g, (((1,), (0,)), ((), ())),
                              preferred_element_type=jnp.float32)
        pieces.append(c_g + off)
        tot = lax.dot_general(ones_row, a_g, (((1,), (0,)), ((), ())),
                              preferred_element_type=jnp.float32)
        off = off + tot
    cnt_before = jnp.concatenate(pieces, axis=0)
    counts = off

    padded = jnp.floor((counts + (TM - 1)) / TM) * TM
    lanes8r = lax.broadcasted_iota(jnp.int32, (NUM_EXPERTS, NUM_EXPERTS), 0)
    lanes8c = lax.broadcasted_iota(jnp.int32, (NUM_EXPERTS, NUM_EXPERTS), 1)
    lincl8 = (lanes8r <= lanes8c).astype(jnp.float32)
    seg_end = lax.dot_general(padded, lincl8, (((1,), (0,)), ((), ())),
                              preferred_element_type=jnp.float32)
    offs = seg_end - padded

    off1 = jnp.sum(offs * oh1, axis=-1, keepdims=True)
    off2 = jnp.sum(offs * oh2, axis=-1, keepdims=True)
    rank1 = jnp.sum(cnt_before * oh1, axis=-1, keepdims=True)
    rank2 = jnp.sum(cnt_before * oh2, axis=-1, keepdims=True)
    slot1 = (off1 + rank1).astype(jnp.int32)
    slot2 = (off2 + rank2).astype(jnp.int32)
    pos_ref[...] = jnp.where(lanes == 0, slot1, jnp.where(lanes == 1, slot2, 0))

    t_start = (lax.broadcasted_iota(jnp.int32, (1, CHUNK), 1) * TM).astype(jnp.float32)
    acc = jnp.zeros((1, CHUNK), jnp.float32)
    lanes8 = lax.broadcasted_iota(jnp.int32, (1, NUM_EXPERTS), 1)
    for e in range(NUM_EXPERTS):
        se = jnp.sum(jnp.where(lanes8 == e, seg_end, 0.0), axis=-1, keepdims=True)
        acc = acc + (se <= t_start).astype(jnp.float32)
    teid = jnp.minimum(acc, NUM_EXPERTS - 1).astype(jnp.int32)
    teid_ref[...] = jnp.broadcast_to(teid, (8, CHUNK))


def _routing(router_logits):
    return pl.pallas_call(
        _routing_body,
        out_shape=(
            jax.ShapeDtypeStruct((TOKENS, NUM_EXPERTS), jnp.int32),
            jax.ShapeDtypeStruct((TOKENS, NUM_EXPERTS), jnp.float32),
            jax.ShapeDtypeStruct((8, CHUNK), jnp.int32),
        ),
    )(router_logits)


def _ffn_body(te_ref, xs_ref, w31_ref, w2_ref, wb_ref, out_ref):
    xs = xs_ref[...]
    proj = lax.dot_general(
        xs, w31_ref[0], (((1,), (1,)), ((), ())), preferred_element_type=jnp.float32
    )
    up = proj[:, :INTER]
    gate = proj[:, INTER:]
    h = gate * jax.nn.sigmoid(gate) * up
    y = lax.dot_general(
        h, w2_ref[0], (((1,), (1,)), ((), ())), preferred_element_type=jnp.float32
    )
    out_ref[...] = y * wb_ref[...]


def _ffn(xs, w3_w1_weight, w2_weight, wbuf, tile_eid):
    grid_spec = pltpu.PrefetchScalarGridSpec(
        num_scalar_prefetch=1,
        grid=(NTILES,),
        in_specs=[
            pl.BlockSpec((TM, HIDDEN), lambda i, te: (i, 0)),
            pl.BlockSpec((1, 2 * INTER, HIDDEN), lambda i, te: (te[i], 0, 0)),
            pl.BlockSpec((1, HIDDEN, INTER), lambda i, te: (te[i], 0, 0)),
            pl.BlockSpec((TM, 1), lambda i, te: (i, 0)),
        ],
        out_specs=pl.BlockSpec((TM, HIDDEN), lambda i, te: (i, 0)),
    )
    return pl.pallas_call(
        _ffn_body,
        grid_spec=grid_spec,
        out_shape=jax.ShapeDtypeStruct((NBUF, HIDDEN), jnp.float32),
        compiler_params=pltpu.CompilerParams(dimension_semantics=("arbitrary",)),
    )(tile_eid, xs, w3_w1_weight, w2_weight, wbuf.reshape(NBUF, 1))


def _sc_dispatch(x, scat_idx):
    info = plsc.get_sparse_core_info()
    nw = info.num_cores * info.num_subcores
    t_per_w = TOKENS // nw
    mesh = plsc.VectorSubcoreMesh(core_axis_name="c", subcore_axis_name="s")

    @functools.partial(
        pl.kernel,
        mesh=mesh,
        out_type=jax.ShapeDtypeStruct((NBUF, HIDDEN), jnp.float32),
        scratch_types=[
            pltpu.VMEM((TOP_K, t_per_w), jnp.int32),
            pltpu.VMEM((t_per_w, HIDDEN), jnp.float32),
            pltpu.SemaphoreType.DMA,
            pltpu.SemaphoreType.DMA,
        ],
    )
    def k(x_hbm, sidx_hbm, xs_hbm, idx_v, rows_v, s0, s1):
        wid = lax.axis_index("s") * info.num_cores + lax.axis_index("c")
        base = wid * t_per_w
        pltpu.sync_copy(sidx_hbm.at[wid], idx_v)
        pltpu.sync_copy(x_hbm.at[pl.ds(base, t_per_w)], rows_v)
        cp0 = pltpu.async_copy(rows_v, xs_hbm.at[idx_v.at[0]], s0)
        cp1 = pltpu.async_copy(rows_v, xs_hbm.at[idx_v.at[1]], s1)
        cp0.wait()
        cp1.wait()

    return k(x, scat_idx)


def _sc_combine(ysw, scat_idx):
    info = plsc.get_sparse_core_info()
    nw = info.num_cores * info.num_subcores
    t_per_w = TOKENS // nw
    chunk = 32
    n_chunks = t_per_w // chunk
    mesh = plsc.VectorSubcoreMesh(core_axis_name="c", subcore_axis_name="s")

    @functools.partial(
        pl.kernel,
        mesh=mesh,
        out_type=jax.ShapeDtypeStruct((TOKENS, HIDDEN), jnp.float32),
        scratch_types=[
            pltpu.VMEM((TOP_K, t_per_w), jnp.int32),
            pltpu.VMEM((chunk, HIDDEN), jnp.float32),
            pltpu.VMEM((chunk, HIDDEN), jnp.float32),
            pltpu.SemaphoreType.DMA,
            pltpu.SemaphoreType.DMA,
        ],
    )
    def k(ysw_hbm, sidx_hbm, out_hbm, idx_v, r0_v, r1_v, s0, s1):
        wid = lax.axis_index("s") * info.num_cores + lax.axis_index("c")
        base = wid * t_per_w
        pltpu.sync_copy(sidx_hbm.at[wid], idx_v)

        def body(c, _):
            off = base + c * chunk
            cp0 = pltpu.async_copy(
                ysw_hbm.at[idx_v.at[0, pl.ds(c * chunk, chunk)]], r0_v, s0)
            cp1 = pltpu.async_copy(
                ysw_hbm.at[idx_v.at[1, pl.ds(c * chunk, chunk)]], r1_v, s1)
            cp0.wait()
            cp1.wait()

            def row(i, _):
                for j in range(HIDDEN // 16):
                    sl = pl.ds(j * 16, 16)
                    r0_v[i, sl] = r0_v[i, sl] + r1_v[i, sl]
                return 0

            lax.fori_loop(0, chunk, row, 0)
            pltpu.sync_copy(r0_v, out_hbm.at[pl.ds(off, chunk)])
            return 0

        lax.fori_loop(0, n_chunks, body, 0)

    return k(ysw, scat_idx)


def kernel(x, router_logits, w3_w1_weight, w2_weight):
    pos_out, w_out, teid_out = _routing(router_logits)
    slot = pos_out[:, :TOP_K].reshape(-1)
    w_flat = w_out[:, :TOP_K].reshape(-1)
    wbuf = jnp.zeros((NBUF,), jnp.float32).at[slot].set(w_flat)
    nw = 32
    scat_idx = pos_out[:, :TOP_K].reshape(nw, TOKENS // nw, TOP_K).transpose(0, 2, 1)
    teid = teid_out[0, :NTILES]
    xs = _sc_dispatch(x, scat_idx)
    ysw = _ffn(xs, w3_w1_weight, w2_weight, wbuf, teid)
    out = _sc_combine(ysw, scat_idx)
    return out.astype(x.dtype)


# speedup vs baseline: 2.3234x; 1.0618x over previous
import functools

import jax
import jax.numpy as jnp
from jax import lax
from jax.experimental import pallas as pl
from jax.experimental.pallas import tpu as pltpu
from jax.experimental.pallas import tpu_sc as plsc

NUM_EXPERTS = 8
TOP_K = 2
HIDDEN = 1024
INTER = 1024
TOKENS = 2048
TM = 256
NASSIGN = TOKENS * TOP_K
NBUF = NASSIGN + NUM_EXPERTS * TM
NTILES = NBUF // TM
CHUNK = 512
TEID_W = 128


def _routing_body(logits_ref, pos_ref, w_ref, teid_ref):
    logits = logits_ref[...]
    m = jnp.max(logits, axis=-1, keepdims=True)
    ex = jnp.exp(logits - m)
    probs = ex / jnp.sum(ex, axis=-1, keepdims=True)
    lanes = lax.broadcasted_iota(jnp.int32, probs.shape, 1)
    m1 = jnp.max(probs, axis=-1, keepdims=True)
    i1 = jnp.min(jnp.where(probs == m1, lanes, NUM_EXPERTS), axis=-1, keepdims=True)
    oh1 = (lanes == i1).astype(jnp.float32)
    masked = jnp.where(lanes == i1, -jnp.inf, probs)
    m2 = jnp.max(masked, axis=-1, keepdims=True)
    i2 = jnp.min(jnp.where(masked == m2, lanes, NUM_EXPERTS), axis=-1, keepdims=True)
    oh2 = (lanes == i2).astype(jnp.float32)
    denom = m1 + m2
    lanes32 = lax.broadcasted_iota(jnp.int32, (TOKENS, 2 * 16), 1)
    w_ref[...] = jnp.where(lanes32 < 16, m1 / denom, m2 / denom)

    a = oh1 + oh2
    rr = lax.broadcasted_iota(jnp.int32, (CHUNK, CHUNK), 0)
    cc = lax.broadcasted_iota(jnp.int32, (CHUNK, CHUNK), 1)
    ltri = (cc < rr).astype(jnp.float32)
    ones_row = jnp.ones((1, CHUNK), jnp.float32)
    n_chunks = TOKENS // CHUNK
    pieces = []
    off = jnp.zeros((1, NUM_EXPERTS), jnp.float32)
    for g in range(n_chunks):
        a_g = a[g * CHUNK:(g + 1) * CHUNK, :]
        c_g = lax.dot_general(ltri, a_g, (((1,), (0,)), ((), ())),
                              preferred_element_type=jnp.float32)
        pieces.append(c_g + off)
        tot = lax.dot_general(ones_row, a_g, (((1,), (0,)), ((), ())),
                              preferred_element_type=jnp.float32)
        off = off + tot
    cnt_before = jnp.concatenate(pieces, axis=0)
    counts = off

    padded = jnp.floor((counts + (TM - 1)) / TM) * TM
    lanes8r = lax.broadcasted_iota(jnp.int32, (NUM_EXPERTS, NUM_EXPERTS), 0)
    lanes8c = lax.broadcasted_iota(jnp.int32, (NUM_EXPERTS, NUM_EXPERTS), 1)
    lincl8 = (lanes8r <= lanes8c).astype(jnp.float32)
    seg_end = lax.dot_general(padded, lincl8, (((1,), (0,)), ((), ())),
                              preferred_element_type=jnp.float32)
    offs = seg_end - padded

    off1 = jnp.sum(offs * oh1, axis=-1, keepdims=True)
    off2 = jnp.sum(offs * oh2, axis=-1, keepdims=True)
    rank1 = jnp.sum(cnt_before * oh1, axis=-1, keepdims=True)
    rank2 = jnp.sum(cnt_before * oh2, axis=-1, keepdims=True)
    slot1 = (off1 + rank1).astype(jnp.int32)
    slot2 = (off2 + rank2).astype(jnp.int32)
    pos_ref[...] = jnp.where(lanes == 0, slot1, jnp.where(lanes == 1, slot2, 0))

    t_start = (lax.broadcasted_iota(jnp.int32, (1, TEID_W), 1) * TM).astype(jnp.float32)
    acc = jnp.zeros((1, TEID_W), jnp.float32)
    lanes8 = lax.broadcasted_iota(jnp.int32, (1, NUM_EXPERTS), 1)
    for e in range(NUM_EXPERTS):
        se = jnp.sum(jnp.where(lanes8 == e, seg_end, 0.0), axis=-1, keepdims=True)
        acc = acc + (se <= t_start).astype(jnp.float32)
    teid = jnp.minimum(acc, NUM_EXPERTS - 1).astype(jnp.int32)
    teid_ref[...] = jnp.broadcast_to(teid, (8, TEID_W))


def _routing(router_logits):
    return pl.pallas_call(
        _routing_body,
        out_shape=(
            jax.ShapeDtypeStruct((TOKENS, NUM_EXPERTS), jnp.int32),
            jax.ShapeDtypeStruct((TOKENS, 2 * 16), jnp.float32),
            jax.ShapeDtypeStruct((8, TEID_W), jnp.int32),
        ),
    )(router_logits)


def _ffn_body(te_ref, xs_ref, w31_ref, w2_ref, out_ref):
    xs = xs_ref[...]
    proj = lax.dot_general(
        xs, w31_ref[0], (((1,), (1,)), ((), ())), preferred_element_type=jnp.float32
    )
    up = proj[:, :INTER]
    gate = proj[:, INTER:]
    h = gate * jax.nn.sigmoid(gate) * up
    out_ref[...] = lax.dot_general(
        h, w2_ref[0], (((1,), (1,)), ((), ())), preferred_element_type=jnp.float32
    )


def _ffn(xs, w3_w1_weight, w2_weight, tile_eid):
    grid_spec = pltpu.PrefetchScalarGridSpec(
        num_scalar_prefetch=1,
        grid=(NTILES,),
        in_specs=[
            pl.BlockSpec((TM, HIDDEN), lambda i, te: (i, 0)),
            pl.BlockSpec((1, 2 * INTER, HIDDEN), lambda i, te: (te[i], 0, 0)),
            pl.BlockSpec((1, HIDDEN, INTER), lambda i, te: (te[i], 0, 0)),
        ],
        out_specs=pl.BlockSpec((TM, HIDDEN), lambda i, te: (i, 0)),
    )
    return pl.pallas_call(
        _ffn_body,
        grid_spec=grid_spec,
        out_shape=jax.ShapeDtypeStruct((NBUF, HIDDEN), jnp.float32),
        compiler_params=pltpu.CompilerParams(dimension_semantics=("arbitrary",)),
    )(tile_eid, xs, w3_w1_weight, w2_weight)


def _sc_dispatch(x, scat_idx):
    info = plsc.get_sparse_core_info()
    nw = info.num_cores * info.num_subcores
    t_per_w = TOKENS // nw
    mesh = plsc.VectorSubcoreMesh(core_axis_name="c", subcore_axis_name="s")

    @functools.partial(
        pl.kernel,
        mesh=mesh,
        out_type=jax.ShapeDtypeStruct((NBUF, HIDDEN), jnp.float32),
        scratch_types=[
            pltpu.VMEM((TOP_K, t_per_w), jnp.int32),
            pltpu.VMEM((t_per_w, HIDDEN), jnp.float32),
            pltpu.SemaphoreType.DMA,
            pltpu.SemaphoreType.DMA,
        ],
    )
    def k(x_hbm, sidx_hbm, xs_hbm, idx_v, rows_v, s0, s1):
        wid = lax.axis_index("s") * info.num_cores + lax.axis_index("c")
        base = wid * t_per_w
        pltpu.sync_copy(sidx_hbm.at[wid], idx_v)
        pltpu.sync_copy(x_hbm.at[pl.ds(base, t_per_w)], rows_v)
        cp0 = pltpu.async_copy(rows_v, xs_hbm.at[idx_v.at[0]], s0)
        cp1 = pltpu.async_copy(rows_v, xs_hbm.at[idx_v.at[1]], s1)
        cp0.wait()
        cp1.wait()

    return k(x, scat_idx)


def _sc_combine(ysw, scat_idx, wmat):
    info = plsc.get_sparse_core_info()
    nw = info.num_cores * info.num_subcores
    t_per_w = TOKENS // nw
    chunk = 32
    n_chunks = t_per_w // chunk
    mesh = plsc.VectorSubcoreMesh(core_axis_name="c", subcore_axis_name="s")

    @functools.partial(
        pl.kernel,
        mesh=mesh,
        out_type=jax.ShapeDtypeStruct((TOKENS, HIDDEN), jnp.float32),
        scratch_types=[
            pltpu.VMEM((TOP_K, t_per_w), jnp.int32),
            pltpu.VMEM((t_per_w, 2 * 16), jnp.float32),
            pltpu.VMEM((chunk, HIDDEN), jnp.float32),
            pltpu.VMEM((chunk, HIDDEN), jnp.float32),
            pltpu.SemaphoreType.DMA,
            pltpu.SemaphoreType.DMA,
        ],
    )
    def k(ysw_hbm, sidx_hbm, wmat_hbm, out_hbm, idx_v, wm_v, r0_v, r1_v, s0, s1):
        wid = lax.axis_index("s") * info.num_cores + lax.axis_index("c")
        base = wid * t_per_w
        pltpu.sync_copy(sidx_hbm.at[wid], idx_v)
        pltpu.sync_copy(wmat_hbm.at[pl.ds(base, t_per_w)], wm_v)

        def body(c, _):
            off = base + c * chunk
            cp0 = pltpu.async_copy(
                ysw_hbm.at[idx_v.at[0, pl.ds(c * chunk, chunk)]], r0_v, s0)
            cp1 = pltpu.async_copy(
                ysw_hbm.at[idx_v.at[1, pl.ds(c * chunk, chunk)]], r1_v, s1)
            cp0.wait()
            cp1.wait()

            def row(i, _):
                w0 = wm_v[c * chunk + i, pl.ds(0, 16)]
                w1 = wm_v[c * chunk + i, pl.ds(16, 16)]
                for j in range(HIDDEN // 16):
                    sl = pl.ds(j * 16, 16)
                    r0_v[i, sl] = r0_v[i, sl] * w0 + r1_v[i, sl] * w1
                return 0

            lax.fori_loop(0, chunk, row, 0)
            pltpu.sync_copy(r0_v, out_hbm.at[pl.ds(off, chunk)])
            return 0

        lax.fori_loop(0, n_chunks, body, 0)

    return k(ysw, scat_idx, wmat)


def kernel(x, router_logits, w3_w1_weight, w2_weight):
    pos_out, wmat, teid_out = _routing(router_logits)
    nw = 32
    scat_idx = pos_out[:, :TOP_K].reshape(nw, TOKENS // nw, TOP_K).transpose(0, 2, 1)
    teid = teid_out[0, :NTILES]
    xs = _sc_dispatch(x, scat_idx)
    ys = _ffn(xs, w3_w1_weight, w2_weight, teid)
    out = _sc_combine(ys, scat_idx, wmat)
    return out.astype(x.dtype)


# speedup vs baseline: 2.5606x; 1.1021x over previous
import functools

import jax
import jax.numpy as jnp
from jax import lax
from jax.experimental import pallas as pl
from jax.experimental.pallas import tpu as pltpu
from jax.experimental.pallas import tpu_sc as plsc

NUM_EXPERTS = 8
TOP_K = 2
HIDDEN = 1024
INTER = 1024
TOKENS = 2048
TM = 256
NASSIGN = TOKENS * TOP_K
NBUF = NASSIGN + NUM_EXPERTS * TM
NTILES = NBUF // TM
CHUNK = 512
TEID_W = 128


def _routing_body(logits_ref, pos_ref, w_ref, teid_ref):
    logits = logits_ref[...]
    m = jnp.max(logits, axis=-1, keepdims=True)
    ex = jnp.exp(logits - m)
    probs = ex / jnp.sum(ex, axis=-1, keepdims=True)
    lanes = lax.broadcasted_iota(jnp.int32, probs.shape, 1)
    m1 = jnp.max(probs, axis=-1, keepdims=True)
    i1 = jnp.min(jnp.where(probs == m1, lanes, NUM_EXPERTS), axis=-1, keepdims=True)
    oh1 = (lanes == i1).astype(jnp.float32)
    masked = jnp.where(lanes == i1, -jnp.inf, probs)
    m2 = jnp.max(masked, axis=-1, keepdims=True)
    i2 = jnp.min(jnp.where(masked == m2, lanes, NUM_EXPERTS), axis=-1, keepdims=True)
    oh2 = (lanes == i2).astype(jnp.float32)
    denom = m1 + m2
    lanes32 = lax.broadcasted_iota(jnp.int32, (TOKENS, 2 * 16), 1)
    w_ref[...] = jnp.where(lanes32 < 16, m1 / denom, m2 / denom)

    a = oh1 + oh2
    rr = lax.broadcasted_iota(jnp.int32, (CHUNK, CHUNK), 0)
    cc = lax.broadcasted_iota(jnp.int32, (CHUNK, CHUNK), 1)
    ltri = (cc < rr).astype(jnp.float32)
    ones_row = jnp.ones((1, CHUNK), jnp.float32)
    n_chunks = TOKENS // CHUNK
    pieces = []
    off = jnp.zeros((1, NUM_EXPERTS), jnp.float32)
    for g in range(n_chunks):
        a_g = a[g * CHUNK:(g + 1) * CHUNK, :]
        c_g = lax.dot_general(ltri, a_g, (((1,), (0,)), ((), ())),
                              preferred_element_type=jnp.float32)
        pieces.append(c_g + off)
        tot = lax.dot_general(ones_row, a_g, (((1,), (0,)), ((), ())),
                              preferred_element_type=jnp.float32)
        off = off + tot
    cnt_before = jnp.concatenate(pieces, axis=0)
    counts = off

    padded = jnp.floor((counts + (TM - 1)) / TM) * TM
    lanes8r = lax.broadcasted_iota(jnp.int32, (NUM_EXPERTS, NUM_EXPERTS), 0)
    lanes8c = lax.broadcasted_iota(jnp.int32, (NUM_EXPERTS, NUM_EXPERTS), 1)
    lincl8 = (lanes8r <= lanes8c).astype(jnp.float32)
    seg_end = lax.dot_general(padded, lincl8, (((1,), (0,)), ((), ())),
                              preferred_element_type=jnp.float32)
    offs = seg_end - padded

    off1 = jnp.sum(offs * oh1, axis=-1, keepdims=True)
    off2 = jnp.sum(offs * oh2, axis=-1, keepdims=True)
    rank1 = jnp.sum(cnt_before * oh1, axis=-1, keepdims=True)
    rank2 = jnp.sum(cnt_before * oh2, axis=-1, keepdims=True)
    slot1 = (off1 + rank1).astype(jnp.int32)
    slot2 = (off2 + rank2).astype(jnp.int32)
    pos_ref[...] = jnp.where(lanes == 0, slot1, jnp.where(lanes == 1, slot2, 0))

    t_start = (lax.broadcasted_iota(jnp.int32, (1, TEID_W), 1) * TM).astype(jnp.float32)
    lanes8 = lax.broadcasted_iota(jnp.int32, (1, NUM_EXPERTS), 1)
    cur = jnp.full((1, TEID_W), -1.0, jnp.float32)
    for e in range(NUM_EXPERTS):
        pe = jnp.sum(jnp.where(lanes8 == e, padded, 0.0), axis=-1, keepdims=True)
        oe = jnp.sum(jnp.where(lanes8 == e, offs, 0.0), axis=-1, keepdims=True)
        cond = jnp.logical_and(pe > 0, oe <= t_start)
        cur = jnp.maximum(cur, jnp.where(cond, float(e), -1.0))
    nxt = jnp.full((1, TEID_W), 99.0, jnp.float32)
    for e in range(NUM_EXPERTS):
        pe = jnp.sum(jnp.where(lanes8 == e, padded, 0.0), axis=-1, keepdims=True)
        cond = jnp.logical_and(pe > 0, cur < float(e))
        nxt = jnp.minimum(nxt, jnp.where(cond, float(e), 99.0))
    hasnxt = (nxt < 99.0).astype(jnp.float32)
    prev = jnp.concatenate(
        [jnp.full((1, 1), -1.0, jnp.float32), cur[:, :TEID_W - 1]], axis=1)
    startf = (cur != prev).astype(jnp.float32)
    rT = lax.broadcasted_iota(jnp.int32, (TEID_W, TEID_W), 0)
    cT = lax.broadcasted_iota(jnp.int32, (TEID_W, TEID_W), 1)
    linclT = (rT <= cT).astype(jnp.float32)
    run_id = lax.dot_general(startf, linclT, (((1,), (0,)), ((), ())),
                             preferred_element_type=jnp.float32) - 1.0
    slot = run_id - 2.0 * jnp.floor(run_id * 0.5)
    zero3 = jnp.zeros((3, TEID_W), jnp.float32)
    teid_ref[...] = jnp.concatenate(
        [cur, slot, startf, jnp.minimum(nxt, 7.0), hasnxt, zero3], axis=0
    ).astype(jnp.int32)


def _routing(router_logits):
    return pl.pallas_call(
        _routing_body,
        out_shape=(
            jax.ShapeDtypeStruct((TOKENS, NUM_EXPERTS), jnp.int32),
            jax.ShapeDtypeStruct((TOKENS, 2 * 16), jnp.float32),
            jax.ShapeDtypeStruct((8, TEID_W), jnp.int32),
        ),
    )(router_logits)


def _ffn_body(te_ref, xs_ref, w31_hbm, w2_hbm, out_ref, w31_v, w2_v, s31, s2):
    i = pl.program_id(0)
    te = te_ref[0, i]
    slot = te_ref[1, i]
    start = te_ref[2, i]
    nxt = te_ref[3, i]
    hasnxt = te_ref[4, i]

    def cp31(e, s):
        return pltpu.make_async_copy(w31_hbm.at[e], w31_v.at[s], s31.at[s])

    def cp2(e, s):
        return pltpu.make_async_copy(w2_hbm.at[e], w2_v.at[s], s2.at[s])

    @pl.when(i == 0)
    def _():
        cp31(te, slot).start()
        cp2(te, slot).start()

    @pl.when(start == 1)
    def _():
        cp31(te, slot).wait()
        cp2(te, slot).wait()

    @pl.when(jnp.logical_and(start == 1, hasnxt == 1))
    def _():
        cp31(nxt, 1 - slot).start()
        cp2(nxt, 1 - slot).start()

    xs = xs_ref[...]
    w31 = w31_v[pl.ds(slot, 1), :, :][0]
    proj = lax.dot_general(
        xs, w31, (((1,), (1,)), ((), ())), preferred_element_type=jnp.float32
    )
    up = proj[:, :INTER]
    gate = proj[:, INTER:]
    h = gate * jax.nn.sigmoid(gate) * up
    w2 = w2_v[pl.ds(slot, 1), :, :][0]
    out_ref[...] = lax.dot_general(
        h, w2, (((1,), (1,)), ((), ())), preferred_element_type=jnp.float32
    )


def _ffn(xs, w3_w1_weight, w2_weight, tile_sched):
    grid_spec = pltpu.PrefetchScalarGridSpec(
        num_scalar_prefetch=1,
        grid=(NTILES,),
        in_specs=[
            pl.BlockSpec((TM, HIDDEN), lambda i, te: (i, 0)),
            pl.BlockSpec(memory_space=pl.ANY),
            pl.BlockSpec(memory_space=pl.ANY),
        ],
        out_specs=pl.BlockSpec((TM, HIDDEN), lambda i, te: (i, 0)),
        scratch_shapes=[
            pltpu.VMEM((2, 2 * INTER, HIDDEN), jnp.float32),
            pltpu.VMEM((2, HIDDEN, INTER), jnp.float32),
            pltpu.SemaphoreType.DMA((2,)),
            pltpu.SemaphoreType.DMA((2,)),
        ],
    )
    return pl.pallas_call(
        _ffn_body,
        grid_spec=grid_spec,
        out_shape=jax.ShapeDtypeStruct((NBUF, HIDDEN), jnp.float32),
        compiler_params=pltpu.CompilerParams(dimension_semantics=("arbitrary",)),
    )(tile_sched, xs, w3_w1_weight, w2_weight)


def _sc_dispatch(x, scat_idx):
    info = plsc.get_sparse_core_info()
    nw = info.num_cores * info.num_subcores
    t_per_w = TOKENS // nw
    mesh = plsc.VectorSubcoreMesh(core_axis_name="c", subcore_axis_name="s")

    @functools.partial(
        pl.kernel,
        mesh=mesh,
        out_type=jax.ShapeDtypeStruct((NBUF, HIDDEN), jnp.float32),
        scratch_types=[
            pltpu.VMEM((TOP_K, t_per_w), jnp.int32),
            pltpu.VMEM((t_per_w, HIDDEN), jnp.float32),
            pltpu.SemaphoreType.DMA,
            pltpu.SemaphoreType.DMA,
        ],
    )
    def k(x_hbm, sidx_hbm, xs_hbm, idx_v, rows_v, s0, s1):
        wid = lax.axis_index("s") * info.num_cores + lax.axis_index("c")
        base = wid * t_per_w
        pltpu.sync_copy(sidx_hbm.at[wid], idx_v)
        pltpu.sync_copy(x_hbm.at[pl.ds(base, t_per_w)], rows_v)
        cp0 = pltpu.async_copy(rows_v, xs_hbm.at[idx_v.at[0]], s0)
        cp1 = pltpu.async_copy(rows_v, xs_hbm.at[idx_v.at[1]], s1)
        cp0.wait()
        cp1.wait()

    return k(x, scat_idx)


def _sc_combine(ysw, scat_idx, wmat):
    info = plsc.get_sparse_core_info()
    nw = info.num_cores * info.num_subcores
    t_per_w = TOKENS // nw
    chunk = 32
    n_chunks = t_per_w // chunk
    mesh = plsc.VectorSubcoreMesh(core_axis_name="c", subcore_axis_name="s")

    @functools.partial(
        pl.kernel,
        mesh=mesh,
        out_type=jax.ShapeDtypeStruct((TOKENS, HIDDEN), jnp.float32),
        scratch_types=[
            pltpu.VMEM((TOP_K, t_per_w), jnp.int32),
            pltpu.VMEM((t_per_w, 2 * 16), jnp.float32),
            pltpu.VMEM((chunk, HIDDEN), jnp.float32),
            pltpu.VMEM((chunk, HIDDEN), jnp.float32),
            pltpu.SemaphoreType.DMA,
            pltpu.SemaphoreType.DMA,
        ],
    )
    def k(ysw_hbm, sidx_hbm, wmat_hbm, out_hbm, idx_v, wm_v, r0_v, r1_v, s0, s1):
        wid = lax.axis_index("s") * info.num_cores + lax.axis_index("c")
        base = wid * t_per_w
        pltpu.sync_copy(sidx_hbm.at[wid], idx_v)
        pltpu.sync_copy(wmat_hbm.at[pl.ds(base, t_per_w)], wm_v)

        def body(c, _):
            off = base + c * chunk
            cp0 = pltpu.async_copy(
                ysw_hbm.at[idx_v.at[0, pl.ds(c * chunk, chunk)]], r0_v, s0)
            cp1 = pltpu.async_copy(
                ysw_hbm.at[idx_v.at[1, pl.ds(c * chunk, chunk)]], r1_v, s1)
            cp0.wait()
            cp1.wait()

            def row(i, _):
                w0 = wm_v[c * chunk + i, pl.ds(0, 16)]
                w1 = wm_v[c * chunk + i, pl.ds(16, 16)]
                for j in range(HIDDEN // 16):
                    sl = pl.ds(j * 16, 16)
                    r0_v[i, sl] = r0_v[i, sl] * w0 + r1_v[i, sl] * w1
                return 0

            lax.fori_loop(0, chunk, row, 0)
            pltpu.sync_copy(r0_v, out_hbm.at[pl.ds(off, chunk)])
            return 0

        lax.fori_loop(0, n_chunks, body, 0)

    return k(ysw, scat_idx, wmat)


def kernel(x, router_logits, w3_w1_weight, w2_weight):
    pos_out, wmat, teid_out = _routing(router_logits)
    nw = 32
    scat_idx = pos_out[:, :TOP_K].reshape(nw, TOKENS // nw, TOP_K).transpose(0, 2, 1)
    xs = _sc_dispatch(x, scat_idx)
    ys = _ffn(xs, w3_w1_weight, w2_weight, teid_out)
    out = _sc_combine(ys, scat_idx, wmat)
    return out.astype(x.dtype)


# speedup vs baseline: 2.5737x; 1.0051x over previous
import functools

import jax
import jax.numpy as jnp
from jax import lax
from jax.experimental import pallas as pl
from jax.experimental.pallas import tpu as pltpu
from jax.experimental.pallas import tpu_sc as plsc

NUM_EXPERTS = 8
TOP_K = 2
HIDDEN = 1024
INTER = 1024
TOKENS = 2048
TM = 256
NASSIGN = TOKENS * TOP_K
NBUF = NASSIGN + NUM_EXPERTS * TM
NTILES = NBUF // TM
CHUNK = 256
TEID_W = 128


def _routing_body(logits_ref, pos_ref, w_ref, teid_ref):
    logits = logits_ref[...]
    m = jnp.max(logits, axis=-1, keepdims=True)
    ex = jnp.exp(logits - m)
    probs = ex / jnp.sum(ex, axis=-1, keepdims=True)
    lanes = lax.broadcasted_iota(jnp.int32, probs.shape, 1)
    m1 = jnp.max(probs, axis=-1, keepdims=True)
    i1 = jnp.min(jnp.where(probs == m1, lanes, NUM_EXPERTS), axis=-1, keepdims=True)
    oh1 = (lanes == i1).astype(jnp.float32)
    masked = jnp.where(lanes == i1, -jnp.inf, probs)
    m2 = jnp.max(masked, axis=-1, keepdims=True)
    i2 = jnp.min(jnp.where(masked == m2, lanes, NUM_EXPERTS), axis=-1, keepdims=True)
    oh2 = (lanes == i2).astype(jnp.float32)
    denom = m1 + m2
    lanes32 = lax.broadcasted_iota(jnp.int32, (TOKENS, 2 * 16), 1)
    w_ref[...] = jnp.where(lanes32 < 16, m1 / denom, m2 / denom)

    a = oh1 + oh2
    rr = lax.broadcasted_iota(jnp.int32, (CHUNK, CHUNK), 0)
    cc = lax.broadcasted_iota(jnp.int32, (CHUNK, CHUNK), 1)
    ltri = (cc < rr).astype(jnp.float32)
    ones_row = jnp.ones((1, CHUNK), jnp.float32)
    n_chunks = TOKENS // CHUNK
    pieces = []
    off = jnp.zeros((1, NUM_EXPERTS), jnp.float32)
    for g in range(n_chunks):
        a_g = a[g * CHUNK:(g + 1) * CHUNK, :]
        c_g = lax.dot_general(ltri, a_g, (((1,), (0,)), ((), ())),
                              preferred_element_type=jnp.float32)
        pieces.append(c_g + off)
        tot = lax.dot_general(ones_row, a_g, (((1,), (0,)), ((), ())),
                              preferred_element_type=jnp.float32)
        off = off + tot
    cnt_before = jnp.concatenate(pieces, axis=0)
    counts = off

    padded = jnp.floor((counts + (TM - 1)) / TM) * TM
    lanes8r = lax.broadcasted_iota(jnp.int32, (NUM_EXPERTS, NUM_EXPERTS), 0)
    lanes8c = lax.broadcasted_iota(jnp.int32, (NUM_EXPERTS, NUM_EXPERTS), 1)
    lincl8 = (lanes8r <= lanes8c).astype(jnp.float32)
    seg_end = lax.dot_general(padded, lincl8, (((1,), (0,)), ((), ())),
                              preferred_element_type=jnp.float32)
    offs = seg_end - padded

    off1 = jnp.sum(offs * oh1, axis=-1, keepdims=True)
    off2 = jnp.sum(offs * oh2, axis=-1, keepdims=True)
    rank1 = jnp.sum(cnt_before * oh1, axis=-1, keepdims=True)
    rank2 = jnp.sum(cnt_before * oh2, axis=-1, keepdims=True)
    slot1 = (off1 + rank1).astype(jnp.int32)
    slot2 = (off2 + rank2).astype(jnp.int32)
    pos_ref[...] = jnp.where(lanes == 0, slot1, jnp.where(lanes == 1, slot2, 0))

    t_start = (lax.broadcasted_iota(jnp.int32, (1, TEID_W), 1) * TM).astype(jnp.float32)
    lanes8 = lax.broadcasted_iota(jnp.int32, (1, NUM_EXPERTS), 1)
    cur = jnp.full((1, TEID_W), -1.0, jnp.float32)
    for e in range(NUM_EXPERTS):
        pe = jnp.sum(jnp.where(lanes8 == e, padded, 0.0), axis=-1, keepdims=True)
        oe = jnp.sum(jnp.where(lanes8 == e, offs, 0.0), axis=-1, keepdims=True)
        cond = jnp.logical_and(pe > 0, oe <= t_start)
        cur = jnp.maximum(cur, jnp.where(cond, float(e), -1.0))
    nxt = jnp.full((1, TEID_W), 99.0, jnp.float32)
    for e in range(NUM_EXPERTS):
        pe = jnp.sum(jnp.where(lanes8 == e, padded, 0.0), axis=-1, keepdims=True)
        cond = jnp.logical_and(pe > 0, cur < float(e))
        nxt = jnp.minimum(nxt, jnp.where(cond, float(e), 99.0))
    hasnxt = (nxt < 99.0).astype(jnp.float32)
    prev = jnp.concatenate(
        [jnp.full((1, 1), -1.0, jnp.float32), cur[:, :TEID_W - 1]], axis=1)
    startf = (cur != prev).astype(jnp.float32)
    rT = lax.broadcasted_iota(jnp.int32, (TEID_W, TEID_W), 0)
    cT = lax.broadcasted_iota(jnp.int32, (TEID_W, TEID_W), 1)
    linclT = (rT <= cT).astype(jnp.float32)
    run_id = lax.dot_general(startf, linclT, (((1,), (0,)), ((), ())),
                             preferred_element_type=jnp.float32) - 1.0
    slot = run_id - 2.0 * jnp.floor(run_id * 0.5)
    zero3 = jnp.zeros((3, TEID_W), jnp.float32)
    teid_ref[...] = jnp.concatenate(
        [cur, slot, startf, jnp.minimum(nxt, 7.0), hasnxt, zero3], axis=0
    ).astype(jnp.int32)


def _routing(router_logits):
    return pl.pallas_call(
        _routing_body,
        out_shape=(
            jax.ShapeDtypeStruct((TOKENS, NUM_EXPERTS), jnp.int32),
            jax.ShapeDtypeStruct((TOKENS, 2 * 16), jnp.float32),
            jax.ShapeDtypeStruct((8, TEID_W), jnp.int32),
        ),
    )(router_logits)


def _ffn_body(te_ref, xs_ref, w31_hbm, w2_hbm, out_ref, w31_v, w2_v, s31, s2):
    i = pl.program_id(0)
    te = te_ref[0, i]
    slot = te_ref[1, i]
    start = te_ref[2, i]
    nxt = te_ref[3, i]
    hasnxt = te_ref[4, i]

    def cp31(e, s):
        return pltpu.make_async_copy(w31_hbm.at[e], w31_v.at[s], s31.at[s])

    def cp2(e, s):
        return pltpu.make_async_copy(w2_hbm.at[e], w2_v.at[s], s2.at[s])

    @pl.when(i == 0)
    def _():
        cp31(te, slot).start()
        cp2(te, slot).start()

    @pl.when(start == 1)
    def _():
        cp31(te, slot).wait()
        cp2(te, slot).wait()

    @pl.when(jnp.logical_and(start == 1, hasnxt == 1))
    def _():
        cp31(nxt, 1 - slot).start()
        cp2(nxt, 1 - slot).start()

    xs = xs_ref[...]
    w31 = w31_v[pl.ds(slot, 1), :, :][0]
    proj = lax.dot_general(
        xs, w31, (((1,), (1,)), ((), ())), preferred_element_type=jnp.float32
    )
    up = proj[:, :INTER]
    gate = proj[:, INTER:]
    h = gate * jax.nn.sigmoid(gate) * up
    w2 = w2_v[pl.ds(slot, 1), :, :][0]
    out_ref[...] = lax.dot_general(
        h, w2, (((1,), (1,)), ((), ())), preferred_element_type=jnp.float32
    )


def _ffn(xs, w3_w1_weight, w2_weight, tile_sched):
    grid_spec = pltpu.PrefetchScalarGridSpec(
        num_scalar_prefetch=1,
        grid=(NTILES,),
        in_specs=[
            pl.BlockSpec((TM, HIDDEN), lambda i, te: (i, 0)),
            pl.BlockSpec(memory_space=pl.ANY),
            pl.BlockSpec(memory_space=pl.ANY),
        ],
        out_specs=pl.BlockSpec((TM, HIDDEN), lambda i, te: (i, 0)),
        scratch_shapes=[
            pltpu.VMEM((2, 2 * INTER, HIDDEN), jnp.float32),
            pltpu.VMEM((2, HIDDEN, INTER), jnp.float32),
            pltpu.SemaphoreType.DMA((2,)),
            pltpu.SemaphoreType.DMA((2,)),
        ],
    )
    return pl.pallas_call(
        _ffn_body,
        grid_spec=grid_spec,
        out_shape=jax.ShapeDtypeStruct((NBUF, HIDDEN), jnp.float32),
        compiler_params=pltpu.CompilerParams(dimension_semantics=("arbitrary",)),
    )(tile_sched, xs, w3_w1_weight, w2_weight)


def _sc_dispatch(x, scat_idx):
    info = plsc.get_sparse_core_info()
    nw = info.num_cores * info.num_subcores
    t_per_w = TOKENS // nw
    mesh = plsc.VectorSubcoreMesh(core_axis_name="c", subcore_axis_name="s")

    @functools.partial(
        pl.kernel,
        mesh=mesh,
        out_type=jax.ShapeDtypeStruct((NBUF, HIDDEN), jnp.float32),
        scratch_types=[
            pltpu.VMEM((TOP_K, t_per_w), jnp.int32),
            pltpu.VMEM((t_per_w, HIDDEN), jnp.float32),
            pltpu.SemaphoreType.DMA,
            pltpu.SemaphoreType.DMA,
        ],
    )
    def k(x_hbm, sidx_hbm, xs_hbm, idx_v, rows_v, s0, s1):
        wid = lax.axis_index("s") * info.num_cores + lax.axis_index("c")
        base = wid * t_per_w
        pltpu.sync_copy(sidx_hbm.at[wid], idx_v)
        pltpu.sync_copy(x_hbm.at[pl.ds(base, t_per_w)], rows_v)
        cp0 = pltpu.async_copy(rows_v, xs_hbm.at[idx_v.at[0]], s0)
        cp1 = pltpu.async_copy(rows_v, xs_hbm.at[idx_v.at[1]], s1)
        cp0.wait()
        cp1.wait()

    return k(x, scat_idx)


def _sc_combine(ysw, scat_idx, wmat):
    info = plsc.get_sparse_core_info()
    nw = info.num_cores * info.num_subcores
    t_per_w = TOKENS // nw
    chunk = 32
    n_chunks = t_per_w // chunk
    mesh = plsc.VectorSubcoreMesh(core_axis_name="c", subcore_axis_name="s")

    @functools.partial(
        pl.kernel,
        mesh=mesh,
        out_type=jax.ShapeDtypeStruct((TOKENS, HIDDEN), jnp.float32),
        scratch_types=[
            pltpu.VMEM((TOP_K, t_per_w), jnp.int32),
            pltpu.VMEM((t_per_w, 2 * 16), jnp.float32),
            pltpu.VMEM((chunk, HIDDEN), jnp.float32),
            pltpu.VMEM((chunk, HIDDEN), jnp.float32),
            pltpu.SemaphoreType.DMA,
            pltpu.SemaphoreType.DMA,
        ],
    )
    def k(ysw_hbm, sidx_hbm, wmat_hbm, out_hbm, idx_v, wm_v, r0_v, r1_v, s0, s1):
        wid = lax.axis_index("s") * info.num_cores + lax.axis_index("c")
        base = wid * t_per_w
        pltpu.sync_copy(sidx_hbm.at[wid], idx_v)
        pltpu.sync_copy(wmat_hbm.at[pl.ds(base, t_per_w)], wm_v)

        def body(c, _):
            off = base + c * chunk
            cp0 = pltpu.async_copy(
                ysw_hbm.at[idx_v.at[0, pl.ds(c * chunk, chunk)]], r0_v, s0)
            cp1 = pltpu.async_copy(
                ysw_hbm.at[idx_v.at[1, pl.ds(c * chunk, chunk)]], r1_v, s1)
            cp0.wait()
            cp1.wait()

            def row(i, _):
                w0 = wm_v[c * chunk + i, pl.ds(0, 16)]
                w1 = wm_v[c * chunk + i, pl.ds(16, 16)]
                for j in range(HIDDEN // 16):
                    sl = pl.ds(j * 16, 16)
                    r0_v[i, sl] = r0_v[i, sl] * w0 + r1_v[i, sl] * w1
                return 0

            lax.fori_loop(0, chunk, row, 0)
            pltpu.sync_copy(r0_v, out_hbm.at[pl.ds(off, chunk)])
            return 0

        lax.fori_loop(0, n_chunks, body, 0)

    return k(ysw, scat_idx, wmat)


def kernel(x, router_logits, w3_w1_weight, w2_weight):
    pos_out, wmat, teid_out = _routing(router_logits)
    nw = 32
    scat_idx = pos_out[:, :TOP_K].reshape(nw, TOKENS // nw, TOP_K).transpose(0, 2, 1)
    xs = _sc_dispatch(x, scat_idx)
    ys = _ffn(xs, w3_w1_weight, w2_weight, teid_out)
    out = _sc_combine(ys, scat_idx, wmat)
    return out.astype(x.dtype)


# speedup vs baseline: 2.5771x; 1.0013x over previous
import functools

import jax
import jax.numpy as jnp
from jax import lax
from jax.experimental import pallas as pl
from jax.experimental.pallas import tpu as pltpu
from jax.experimental.pallas import tpu_sc as plsc

NUM_EXPERTS = 8
TOP_K = 2
HIDDEN = 1024
INTER = 1024
TOKENS = 2048
TM = 256
NASSIGN = TOKENS * TOP_K
NBUF = NASSIGN + NUM_EXPERTS * TM
NTILES = NBUF // TM
CHUNK = 256
TEID_W = 128


def _routing_body(logits_ref, pos_ref, w_ref, teid_ref):
    logits = logits_ref[...]
    m = jnp.max(logits, axis=-1, keepdims=True)
    ex = jnp.exp(logits - m)
    probs = ex / jnp.sum(ex, axis=-1, keepdims=True)
    lanes = lax.broadcasted_iota(jnp.int32, probs.shape, 1)
    m1 = jnp.max(probs, axis=-1, keepdims=True)
    i1 = jnp.min(jnp.where(probs == m1, lanes, NUM_EXPERTS), axis=-1, keepdims=True)
    oh1 = (lanes == i1).astype(jnp.float32)
    masked = jnp.where(lanes == i1, -jnp.inf, probs)
    m2 = jnp.max(masked, axis=-1, keepdims=True)
    i2 = jnp.min(jnp.where(masked == m2, lanes, NUM_EXPERTS), axis=-1, keepdims=True)
    oh2 = (lanes == i2).astype(jnp.float32)
    denom = m1 + m2
    lanes32 = lax.broadcasted_iota(jnp.int32, (TOKENS, 2 * 16), 1)
    w_ref[...] = jnp.where(lanes32 < 16, m1 / denom, m2 / denom)

    a = oh1 + oh2
    rr = lax.broadcasted_iota(jnp.int32, (CHUNK, CHUNK), 0)
    cc = lax.broadcasted_iota(jnp.int32, (CHUNK, CHUNK), 1)
    ltri = (cc < rr).astype(jnp.float32)
    ones_row = jnp.ones((1, CHUNK), jnp.float32)
    n_chunks = TOKENS // CHUNK
    pieces = []
    off = jnp.zeros((1, NUM_EXPERTS), jnp.float32)
    for g in range(n_chunks):
        a_g = a[g * CHUNK:(g + 1) * CHUNK, :]
        c_g = lax.dot_general(ltri, a_g, (((1,), (0,)), ((), ())),
                              preferred_element_type=jnp.float32)
        pieces.append(c_g + off)
        tot = lax.dot_general(ones_row, a_g, (((1,), (0,)), ((), ())),
                              preferred_element_type=jnp.float32)
        off = off + tot
    cnt_before = jnp.concatenate(pieces, axis=0)
    counts = off

    padded = jnp.floor((counts + (TM - 1)) / TM) * TM
    lanes8r = lax.broadcasted_iota(jnp.int32, (NUM_EXPERTS, NUM_EXPERTS), 0)
    lanes8c = lax.broadcasted_iota(jnp.int32, (NUM_EXPERTS, NUM_EXPERTS), 1)
    lincl8 = (lanes8r <= lanes8c).astype(jnp.float32)
    seg_end = lax.dot_general(padded, lincl8, (((1,), (0,)), ((), ())),
                              preferred_element_type=jnp.float32)
    offs = seg_end - padded

    off1 = jnp.sum(offs * oh1, axis=-1, keepdims=True)
    off2 = jnp.sum(offs * oh2, axis=-1, keepdims=True)
    rank1 = jnp.sum(cnt_before * oh1, axis=-1, keepdims=True)
    rank2 = jnp.sum(cnt_before * oh2, axis=-1, keepdims=True)
    slot1 = (off1 + rank1).astype(jnp.int32)
    slot2 = (off2 + rank2).astype(jnp.int32)
    pos_ref[...] = jnp.where(lanes == 0, slot1, jnp.where(lanes == 1, slot2, 0))

    t_start = (lax.broadcasted_iota(jnp.int32, (1, TEID_W), 1) * TM).astype(jnp.float32)
    lanes8 = lax.broadcasted_iota(jnp.int32, (1, NUM_EXPERTS), 1)
    cur = jnp.full((1, TEID_W), -1.0, jnp.float32)
    for e in range(NUM_EXPERTS):
        pe = jnp.sum(jnp.where(lanes8 == e, padded, 0.0), axis=-1, keepdims=True)
        oe = jnp.sum(jnp.where(lanes8 == e, offs, 0.0), axis=-1, keepdims=True)
        cond = jnp.logical_and(pe > 0, oe <= t_start)
        cur = jnp.maximum(cur, jnp.where(cond, float(e), -1.0))
    nxt = jnp.full((1, TEID_W), 99.0, jnp.float32)
    for e in range(NUM_EXPERTS):
        pe = jnp.sum(jnp.where(lanes8 == e, padded, 0.0), axis=-1, keepdims=True)
        cond = jnp.logical_and(pe > 0, cur < float(e))
        nxt = jnp.minimum(nxt, jnp.where(cond, float(e), 99.0))
    hasnxt = (nxt < 99.0).astype(jnp.float32)
    prev = jnp.concatenate(
        [jnp.full((1, 1), -1.0, jnp.float32), cur[:, :TEID_W - 1]], axis=1)
    startf = (cur != prev).astype(jnp.float32)
    rT = lax.broadcasted_iota(jnp.int32, (TEID_W, TEID_W), 0)
    cT = lax.broadcasted_iota(jnp.int32, (TEID_W, TEID_W), 1)
    linclT = (rT <= cT).astype(jnp.float32)
    run_id = lax.dot_general(startf, linclT, (((1,), (0,)), ((), ())),
                             preferred_element_type=jnp.float32) - 1.0
    slot = run_id - 2.0 * jnp.floor(run_id * 0.5)
    zero3 = jnp.zeros((3, TEID_W), jnp.float32)
    teid_ref[...] = jnp.concatenate(
        [cur, slot, startf, jnp.minimum(nxt, 7.0), hasnxt, zero3], axis=0
    ).astype(jnp.int32)


def _routing(router_logits):
    return pl.pallas_call(
        _routing_body,
        out_shape=(
            jax.ShapeDtypeStruct((TOKENS, NUM_EXPERTS), jnp.int32),
            jax.ShapeDtypeStruct((TOKENS, 2 * 16), jnp.float32),
            jax.ShapeDtypeStruct((8, TEID_W), jnp.int32),
        ),
    )(router_logits)


def _ffn_body(te_ref, xs_ref, w31_hbm, w2_hbm, out_ref, w31_v, w2_v, s31, s2):
    i = pl.program_id(0)
    te = te_ref[0, i]
    slot = te_ref[1, i]
    start = te_ref[2, i]
    nxt = te_ref[3, i]
    hasnxt = te_ref[4, i]

    def cp31(e, s):
        return pltpu.make_async_copy(w31_hbm.at[e], w31_v.at[s], s31.at[s])

    def cp2(e, s):
        return pltpu.make_async_copy(w2_hbm.at[e], w2_v.at[s], s2.at[s])

    @pl.when(i == 0)
    def _():
        cp31(te, slot).start()
        cp2(te, slot).start()

    @pl.when(start == 1)
    def _():
        cp31(te, slot).wait()
        cp2(te, slot).wait()

    @pl.when(jnp.logical_and(start == 1, hasnxt == 1))
    def _():
        cp31(nxt, 1 - slot).start()
        cp2(nxt, 1 - slot).start()

    xs = xs_ref[...]
    w31 = w31_v[pl.ds(slot, 1), :, :][0]
    proj = lax.dot_general(
        xs, w31, (((1,), (1,)), ((), ())), preferred_element_type=jnp.float32
    )
    up = proj[:, :INTER]
    gate = proj[:, INTER:]
    h = gate * jax.nn.sigmoid(gate) * up
    w2 = w2_v[pl.ds(slot, 1), :, :][0]
    out_ref[...] = lax.dot_general(
        h, w2, (((1,), (1,)), ((), ())), preferred_element_type=jnp.float32
    )


def _ffn(xs, w3_w1_weight, w2_weight, tile_sched):
    grid_spec = pltpu.PrefetchScalarGridSpec(
        num_scalar_prefetch=1,
        grid=(NTILES,),
        in_specs=[
            pl.BlockSpec((TM, HIDDEN), lambda i, te: (i, 0)),
            pl.BlockSpec(memory_space=pl.ANY),
            pl.BlockSpec(memory_space=pl.ANY),
        ],
        out_specs=pl.BlockSpec((TM, HIDDEN), lambda i, te: (i, 0)),
        scratch_shapes=[
            pltpu.VMEM((2, 2 * INTER, HIDDEN), jnp.float32),
            pltpu.VMEM((2, HIDDEN, INTER), jnp.float32),
            pltpu.SemaphoreType.DMA((2,)),
            pltpu.SemaphoreType.DMA((2,)),
        ],
    )
    return pl.pallas_call(
        _ffn_body,
        grid_spec=grid_spec,
        out_shape=jax.ShapeDtypeStruct((NBUF, HIDDEN), jnp.float32),
        compiler_params=pltpu.CompilerParams(dimension_semantics=("arbitrary",)),
    )(tile_sched, xs, w3_w1_weight, w2_weight)


def _sc_dispatch(x, scat_idx):
    info = plsc.get_sparse_core_info()
    nw = info.num_cores * info.num_subcores
    t_per_w = TOKENS // nw
    mesh = plsc.VectorSubcoreMesh(core_axis_name="c", subcore_axis_name="s")

    @functools.partial(
        pl.kernel,
        mesh=mesh,
        out_type=jax.ShapeDtypeStruct((NBUF, HIDDEN), jnp.float32),
        scratch_types=[
            pltpu.VMEM((TOP_K, t_per_w), jnp.int32),
            pltpu.VMEM((t_per_w, HIDDEN), jnp.float32),
            pltpu.SemaphoreType.DMA,
            pltpu.SemaphoreType.DMA,
            pltpu.SemaphoreType.DMA,
            pltpu.SemaphoreType.DMA,
        ],
    )
    def k(x_hbm, sidx_hbm, xs_hbm, idx_v, rows_v, s0, s1, s2, s3):
        wid = lax.axis_index("s") * info.num_cores + lax.axis_index("c")
        base = wid * t_per_w
        cpi = pltpu.async_copy(sidx_hbm.at[wid], idx_v, s2)
        cpx = pltpu.async_copy(x_hbm.at[pl.ds(base, t_per_w)], rows_v, s3)
        cpi.wait()
        cpx.wait()
        cp0 = pltpu.async_copy(rows_v, xs_hbm.at[idx_v.at[0]], s0)
        cp1 = pltpu.async_copy(rows_v, xs_hbm.at[idx_v.at[1]], s1)
        cp0.wait()
        cp1.wait()

    return k(x, scat_idx)


def _sc_combine(ysw, scat_idx, wmat):
    info = plsc.get_sparse_core_info()
    nw = info.num_cores * info.num_subcores
    t_per_w = TOKENS // nw
    chunk = 32
    n_chunks = t_per_w // chunk
    mesh = plsc.VectorSubcoreMesh(core_axis_name="c", subcore_axis_name="s")

    @functools.partial(
        pl.kernel,
        mesh=mesh,
        out_type=jax.ShapeDtypeStruct((TOKENS, HIDDEN), jnp.float32),
        scratch_types=[
            pltpu.VMEM((TOP_K, t_per_w), jnp.int32),
            pltpu.VMEM((t_per_w, 2 * 16), jnp.float32),
            pltpu.VMEM((chunk, HIDDEN), jnp.float32),
            pltpu.VMEM((chunk, HIDDEN), jnp.float32),
            pltpu.SemaphoreType.DMA,
            pltpu.SemaphoreType.DMA,
        ],
    )
    def k(ysw_hbm, sidx_hbm, wmat_hbm, out_hbm, idx_v, wm_v, r0_v, r1_v, s0, s1):
        wid = lax.axis_index("s") * info.num_cores + lax.axis_index("c")
        base = wid * t_per_w
        pltpu.sync_copy(sidx_hbm.at[wid], idx_v)
        pltpu.sync_copy(wmat_hbm.at[pl.ds(base, t_per_w)], wm_v)

        def body(c, _):
            off = base + c * chunk
            cp0 = pltpu.async_copy(
                ysw_hbm.at[idx_v.at[0, pl.ds(c * chunk, chunk)]], r0_v, s0)
            cp1 = pltpu.async_copy(
                ysw_hbm.at[idx_v.at[1, pl.ds(c * chunk, chunk)]], r1_v, s1)
            cp0.wait()
            cp1.wait()

            def row(i, _):
                w0 = wm_v[c * chunk + i, pl.ds(0, 16)]
                w1 = wm_v[c * chunk + i, pl.ds(16, 16)]
                for j in range(HIDDEN // 16):
                    sl = pl.ds(j * 16, 16)
                    r0_v[i, sl] = r0_v[i, sl] * w0 + r1_v[i, sl] * w1
                return 0

            lax.fori_loop(0, chunk, row, 0)
            pltpu.sync_copy(r0_v, out_hbm.at[pl.ds(off, chunk)])
            return 0

        lax.fori_loop(0, n_chunks, body, 0)

    return k(ysw, scat_idx, wmat)


def kernel(x, router_logits, w3_w1_weight, w2_weight):
    pos_out, wmat, teid_out = _routing(router_logits)
    nw = 32
    scat_idx = pos_out[:, :TOP_K].reshape(nw, TOKENS // nw, TOP_K).transpose(0, 2, 1)
    xs = _sc_dispatch(x, scat_idx)
    ys = _ffn(xs, w3_w1_weight, w2_weight, teid_out)
    out = _sc_combine(ys, scat_idx, wmat)
    return out.astype(x.dtype)
